# baseline (speedup 1.0000x reference)
import functools

import jax
import jax.numpy as jnp
from jax import lax
from jax.experimental import pallas as pl
from jax.experimental.pallas import tpu as pltpu
from jax.experimental.pallas import tpu_sc as plsc

_BT = 64


def _gate_route(x, Wg, bg):
    T, D = x.shape
    En = Wg.shape[1]
    TB = 256
    nblk = T // TB

    def body(x_ref, wg_ref, bg_ref, e_ref, r_ref, c_ref, run_ref):
        i = pl.program_id(0)

        @pl.when(i == 0)
        def _():
            run_ref[...] = jnp.zeros_like(run_ref)

        logits = jnp.dot(x_ref[...], wg_ref[...],
                         preferred_element_type=jnp.float32)
        logits = logits + bg_ref[0]
        am = jnp.argmax(logits, axis=-1).astype(jnp.int32)
        col = lax.broadcasted_iota(jnp.int32, (TB, En), 1)
        oh = (am[:, None] == col).astype(jnp.float32)
        row_i = lax.broadcasted_iota(jnp.int32, (TB, TB), 0)
        col_i = lax.broadcasted_iota(jnp.int32, (TB, TB), 1)
        lt = (col_i < row_i).astype(jnp.float32)
        prefix = jnp.dot(lt, oh, preferred_element_type=jnp.float32)
        rank_in = jnp.sum(oh * prefix, axis=1)
        base = jnp.sum(oh * run_ref[0][None, :], axis=1)
        e_ref[0, 0, :] = am
        r_ref[0, 0, :] = (rank_in + base).astype(jnp.int32)
        run_ref[...] = run_ref[...] + jnp.sum(oh, axis=0)[None, :]
        c_ref[...] = run_ref[...].astype(jnp.int32)

    e_o, r_o, c_o = pl.pallas_call(
        body,
        grid=(nblk,),
        in_specs=[
            pl.BlockSpec((TB, D), lambda i: (i, 0)),
            pl.BlockSpec((D, En), lambda i: (0, 0)),
            pl.BlockSpec((1, En), lambda i: (0, 0)),
        ],
        out_specs=[
            pl.BlockSpec((1, 1, TB), lambda i: (i, 0, 0)),
            pl.BlockSpec((1, 1, TB), lambda i: (i, 0, 0)),
            pl.BlockSpec((1, En), lambda i: (0, 0)),
        ],
        out_shape=[
            jax.ShapeDtypeStruct((nblk, 1, TB), jnp.int32),
            jax.ShapeDtypeStruct((nblk, 1, TB), jnp.int32),
            jax.ShapeDtypeStruct((1, En), jnp.int32),
        ],
        scratch_shapes=[pltpu.VMEM((1, En), jnp.float32)],
    )(x, Wg, bg.reshape(1, En))
    return e_o, r_o, c_o


def _pack_route(e_o, r_o, c_o, nbmax_pad):
    nblk, _, TB = e_o.shape
    En = c_o.shape[1]

    def body(c_ref, e_ref, r_ref, p_ref, be_ref, nbt_ref, pad_ref):
        i = pl.program_id(0)

        @pl.when(i == 0)
        def _():
            counts = c_ref[0].astype(jnp.float32)
            nb = jnp.floor((counts + (_BT - 1)) * (1.0 / _BT))
            row_i = lax.broadcasted_iota(jnp.int32, (En, En), 0)
            col_i = lax.broadcasted_iota(jnp.int32, (En, En), 1)
            ut = (row_i <= col_i).astype(jnp.float32)
            cum = jnp.dot(nb[None, :], ut,
                          preferred_element_type=jnp.float32)[0]
            pad_ref[0, :] = (cum - nb) * float(_BT)
            nbt = cum[En - 1]
            nbt_ref[...] = jnp.broadcast_to(nbt.astype(jnp.int32), (1, 1))
            blk = lax.broadcasted_iota(
                jnp.int32, (nbmax_pad, 1), 0).astype(jnp.float32)
            blk = jnp.minimum(blk, nbt - 1.0)
            cmp = (cum[None, :] <= blk).astype(jnp.float32)
            be_ref[0, 0, :] = jnp.sum(cmp, axis=1).astype(jnp.int32)

        am = e_ref[0, 0, :]
        col = lax.broadcasted_iota(jnp.int32, (TB, En), 1)
        oh = (am[:, None] == col).astype(jnp.float32)
        base = jnp.sum(oh * pad_ref[0][None, :], axis=1).astype(jnp.int32)
        p_ref[0, 0, :] = base + r_ref[0, 0, :]

    p_o, be_o, nbt_o = pl.pallas_call(
        body,
        grid=(nblk,),
        in_specs=[
            pl.BlockSpec((1, En), lambda i: (0, 0)),
            pl.BlockSpec((1, 1, TB), lambda i: (i, 0, 0)),
            pl.BlockSpec((1, 1, TB), lambda i: (i, 0, 0)),
        ],
        out_specs=[
            pl.BlockSpec((1, 1, TB), lambda i: (i, 0, 0)),
            pl.BlockSpec((1, 1, nbmax_pad), lambda i: (0, 0, 0)),
            pl.BlockSpec((1, 1), lambda i: (0, 0)),
        ],
        out_shape=[
            jax.ShapeDtypeStruct((nblk, 1, TB), jnp.int32),
            jax.ShapeDtypeStruct((1, 1, nbmax_pad), jnp.int32),
            jax.ShapeDtypeStruct((1, 1), jnp.int32),
        ],
        scratch_shapes=[pltpu.VMEM((1, En), jnp.float32)],
    )(c_o, e_o, r_o)
    return p_o, be_o.reshape(nbmax_pad), nbt_o.reshape(1)


def _row_scatter(x, inv3, P):
    nw, n_it, ch = inv3.shape
    D = x.shape[1]
    per_w = n_it * ch
    info = plsc.get_sparse_core_info()
    assert nw == info.num_cores * info.num_subcores
    mesh = plsc.VectorSubcoreMesh(core_axis_name="c", subcore_axis_name="s")

    @functools.partial(
        pl.kernel,
        mesh=mesh,
        out_type=jax.ShapeDtypeStruct((P, D), x.dtype),
        scratch_types=[
            pltpu.VMEM((n_it, ch), jnp.int32),
            pltpu.VMEM((ch, D), x.dtype),
            pltpu.VMEM((ch, D), x.dtype),
            pltpu.SemaphoreType.DMA,
            pltpu.SemaphoreType.DMA,
        ],
    )
    def k(x_hbm, inv_hbm, out_hbm, idx_v, rows0, rows1, lsem, ssem):
        wid = lax.axis_index("s") * info.num_cores + lax.axis_index("c")
        base = wid * per_w
        pltpu.sync_copy(inv_hbm.at[wid], idx_v)
        bufs = (rows0, rows1)
        loads = [None] * n_it
        scats = [None] * n_it
        loads[0] = pltpu.async_copy(
            x_hbm.at[pl.ds(base, ch)], bufs[0], lsem)
        for i in range(n_it):
            cur = bufs[i % 2]
            loads[i].wait()
            if i + 1 < n_it:
                if i >= 1:
                    scats[i - 1].wait()
                loads[i + 1] = pltpu.async_copy(
                    x_hbm.at[pl.ds(base + (i + 1) * ch, ch)],
                    bufs[(i + 1) % 2], lsem)
            scats[i] = pltpu.async_copy(
                cur, out_hbm.at[idx_v.at[i]], ssem)
        scats[n_it - 1].wait()
        if n_it >= 2:
            scats[n_it - 2].wait()

    return k(x, inv3)


def _row_gather(table, idx):
    R = idx.shape[0]
    D = table.shape[1]
    info = plsc.get_sparse_core_info()
    nw = info.num_cores * info.num_subcores
    per_w = R // nw
    ch = 48 if per_w % 48 == 0 else (32 if per_w % 32 == 0 else per_w)
    n_it = per_w // ch
    mesh = plsc.VectorSubcoreMesh(core_axis_name="c", subcore_axis_name="s")

    @functools.partial(
        pl.kernel,
        mesh=mesh,
        out_type=jax.ShapeDtypeStruct((R, D), table.dtype),
        scratch_types=[
            pltpu.VMEM((per_w,), jnp.int32),
            pltpu.VMEM((ch, D), table.dtype),
            pltpu.VMEM((ch, D), table.dtype),
            pltpu.SemaphoreType.DMA,
            pltpu.SemaphoreType.DMA,
        ],
    )
    def k(table_hbm, idx_hbm, out_hbm, idx_v, rows0, rows1, gsem, ssem):
        wid = lax.axis_index("s") * info.num_cores + lax.axis_index("c")
        base = wid * per_w
        pltpu.sync_copy(idx_hbm.at[pl.ds(base, per_w)], idx_v)
        bufs = (rows0, rows1)
        gathers = [None] * n_it
        stores = [None] * n_it
        gathers[0] = pltpu.async_copy(
            table_hbm.at[idx_v.at[pl.ds(0, ch)]], bufs[0], gsem)
        for i in range(n_it):
            cur = bufs[i % 2]
            gathers[i].wait()
            if i + 1 < n_it:
                if i >= 1:
                    stores[i - 1].wait()
                gathers[i + 1] = pltpu.async_copy(
                    table_hbm.at[idx_v.at[pl.ds((i + 1) * ch, ch)]],
                    bufs[(i + 1) % 2], gsem)
            stores[i] = pltpu.async_copy(
                cur, out_hbm.at[pl.ds(base + i * ch, ch)], ssem)
        stores[n_it - 1].wait()
        if n_it >= 2:
            stores[n_it - 2].wait()

    return k(table, idx)


def _grouped_ffn(xs, W1, b1, W2, b2, block_expert, nb_tot):
    P, D = xs.shape
    F = W1.shape[2]

    def body(be_ref, nbt_ref, xs_ref, w1_ref, b1_ref, w2_ref, b2_ref, o_ref):
        @pl.when(pl.program_id(0) < nbt_ref[0])
        def _():
            h = jnp.dot(xs_ref[...], w1_ref[0],
                        preferred_element_type=jnp.float32) + b1_ref[0, 0]
            h = jax.nn.gelu(h)
            o_ref[...] = jnp.dot(h, w2_ref[0],
                                 preferred_element_type=jnp.float32) + b2_ref[0, 0]

    grid_spec = pltpu.PrefetchScalarGridSpec(
        num_scalar_prefetch=2,
        grid=(P // _BT,),
        in_specs=[
            pl.BlockSpec((_BT, D), lambda i, be, nbt: (i, 0)),
            pl.BlockSpec((1, D, F), lambda i, be, nbt: (be[i], 0, 0)),
            pl.BlockSpec((1, 1, F), lambda i, be, nbt: (be[i], 0, 0)),
            pl.BlockSpec((1, F, D), lambda i, be, nbt: (be[i], 0, 0)),
            pl.BlockSpec((1, 1, D), lambda i, be, nbt: (be[i], 0, 0)),
        ],
        out_specs=pl.BlockSpec((_BT, D), lambda i, be, nbt: (i, 0)),
    )
    En = W1.shape[0]
    return pl.pallas_call(
        body,
        grid_spec=grid_spec,
        out_shape=jax.ShapeDtypeStruct((P, D), jnp.float32),
    )(block_expert, nb_tot, xs, W1, b1.reshape(En, 1, F), W2,
      b2.reshape(En, 1, D))


def kernel(x, Wg, bg, W1, b1, W2, b2):
    T, D = x.shape
    En = Wg.shape[1]
    nbmax = T // _BT + En
    P = nbmax * _BT
    info = plsc.get_sparse_core_info()
    nw = info.num_cores * info.num_subcores
    ch = 32
    e_o, r_o, c_o = _gate_route(x, Wg, bg)
    p_o, block_expert, nb_tot = _pack_route(e_o, r_o, c_o, 128)
    inv = p_o.reshape(T)
    inv3 = p_o.reshape(nw, T // (nw * ch), ch)
    xs = _row_scatter(x, inv3, P)
    ys = _grouped_ffn(xs, W1, b1, W2, b2, block_expert, nb_tot)
    return _row_gather(ys, inv)


# speedup vs baseline: 1.1704x; 1.1704x over previous
import functools

import jax
import jax.numpy as jnp
from jax import lax
from jax.experimental import pallas as pl
from jax.experimental.pallas import tpu as pltpu
from jax.experimental.pallas import tpu_sc as plsc

_BT = 128


def _gate_route(x, Wg, bg):
    T, D = x.shape
    En = Wg.shape[1]
    TB = 256
    nblk = T // TB

    def body(x_ref, wg_ref, bg_ref, e_ref, r_ref, c_ref, run_ref):
        i = pl.program_id(0)

        @pl.when(i == 0)
        def _():
            run_ref[...] = jnp.zeros_like(run_ref)

        logits = jnp.dot(x_ref[...], wg_ref[...],
                         preferred_element_type=jnp.float32)
        logits = logits + bg_ref[0]
        am = jnp.argmax(logits, axis=-1).astype(jnp.int32)
        col = lax.broadcasted_iota(jnp.int32, (TB, En), 1)
        oh = (am[:, None] == col).astype(jnp.float32)
        row_i = lax.broadcasted_iota(jnp.int32, (TB, TB), 0)
        col_i = lax.broadcasted_iota(jnp.int32, (TB, TB), 1)
        lt = (col_i < row_i).astype(jnp.float32)
        prefix = jnp.dot(lt, oh, preferred_element_type=jnp.float32)
        rank_in = jnp.sum(oh * prefix, axis=1)
        base = jnp.sum(oh * run_ref[0][None, :], axis=1)
        e_ref[0, 0, :] = am
        r_ref[0, 0, :] = (rank_in + base).astype(jnp.int32)
        run_ref[...] = run_ref[...] + jnp.sum(oh, axis=0)[None, :]
        c_ref[...] = run_ref[...].astype(jnp.int32)

    e_o, r_o, c_o = pl.pallas_call(
        body,
        grid=(nblk,),
        in_specs=[
            pl.BlockSpec((TB, D), lambda i: (i, 0)),
            pl.BlockSpec((D, En), lambda i: (0, 0)),
            pl.BlockSpec((1, En), lambda i: (0, 0)),
        ],
        out_specs=[
            pl.BlockSpec((1, 1, TB), lambda i: (i, 0, 0)),
            pl.BlockSpec((1, 1, TB), lambda i: (i, 0, 0)),
            pl.BlockSpec((1, En), lambda i: (0, 0)),
        ],
        out_shape=[
            jax.ShapeDtypeStruct((nblk, 1, TB), jnp.int32),
            jax.ShapeDtypeStruct((nblk, 1, TB), jnp.int32),
            jax.ShapeDtypeStruct((1, En), jnp.int32),
        ],
        scratch_shapes=[pltpu.VMEM((1, En), jnp.float32)],
    )(x, Wg, bg.reshape(1, En))
    return e_o, r_o, c_o


def _pack_route(e_o, r_o, c_o, nbmax_pad):
    nblk, _, TB = e_o.shape
    En = c_o.shape[1]

    def body(c_ref, e_ref, r_ref, p_ref, be_ref, nbt_ref, pad_ref):
        i = pl.program_id(0)

        @pl.when(i == 0)
        def _():
            counts = c_ref[0].astype(jnp.float32)
            nb = jnp.floor((counts + (_BT - 1)) * (1.0 / _BT))
            row_i = lax.broadcasted_iota(jnp.int32, (En, En), 0)
            col_i = lax.broadcasted_iota(jnp.int32, (En, En), 1)
            ut = (row_i <= col_i).astype(jnp.float32)
            cum = jnp.dot(nb[None, :], ut,
                          preferred_element_type=jnp.float32)[0]
            pad_ref[0, :] = (cum - nb) * float(_BT)
            nbt = cum[En - 1]
            nbt_ref[...] = jnp.broadcast_to(nbt.astype(jnp.int32), (1, 1))
            blk = lax.broadcasted_iota(
                jnp.int32, (nbmax_pad, 1), 0).astype(jnp.float32)
            blk = jnp.minimum(blk, nbt - 1.0)
            cmp = (cum[None, :] <= blk).astype(jnp.float32)
            be_ref[0, 0, :] = jnp.sum(cmp, axis=1).astype(jnp.int32)

        am = e_ref[0, 0, :]
        col = lax.broadcasted_iota(jnp.int32, (TB, En), 1)
        oh = (am[:, None] == col).astype(jnp.float32)
        base = jnp.sum(oh * pad_ref[0][None, :], axis=1).astype(jnp.int32)
        p_ref[0, 0, :] = base + r_ref[0, 0, :]

    p_o, be_o, nbt_o = pl.pallas_call(
        body,
        grid=(nblk,),
        in_specs=[
            pl.BlockSpec((1, En), lambda i: (0, 0)),
            pl.BlockSpec((1, 1, TB), lambda i: (i, 0, 0)),
            pl.BlockSpec((1, 1, TB), lambda i: (i, 0, 0)),
        ],
        out_specs=[
            pl.BlockSpec((1, 1, TB), lambda i: (i, 0, 0)),
            pl.BlockSpec((1, 1, nbmax_pad), lambda i: (0, 0, 0)),
            pl.BlockSpec((1, 1), lambda i: (0, 0)),
        ],
        out_shape=[
            jax.ShapeDtypeStruct((nblk, 1, TB), jnp.int32),
            jax.ShapeDtypeStruct((1, 1, nbmax_pad), jnp.int32),
            jax.ShapeDtypeStruct((1, 1), jnp.int32),
        ],
        scratch_shapes=[pltpu.VMEM((1, En), jnp.float32)],
    )(c_o, e_o, r_o)
    return p_o, be_o.reshape(nbmax_pad), nbt_o.reshape(1)


def _row_scatter(x, inv3, P):
    nw, n_it, ch = inv3.shape
    D = x.shape[1]
    per_w = n_it * ch
    info = plsc.get_sparse_core_info()
    assert nw == info.num_cores * info.num_subcores
    mesh = plsc.VectorSubcoreMesh(core_axis_name="c", subcore_axis_name="s")

    @functools.partial(
        pl.kernel,
        mesh=mesh,
        out_type=jax.ShapeDtypeStruct((P, D), x.dtype),
        scratch_types=[
            pltpu.VMEM((n_it, ch), jnp.int32),
            pltpu.VMEM((ch, D), x.dtype),
            pltpu.VMEM((ch, D), x.dtype),
            pltpu.SemaphoreType.DMA,
            pltpu.SemaphoreType.DMA,
        ],
    )
    def k(x_hbm, inv_hbm, out_hbm, idx_v, rows0, rows1, lsem, ssem):
        wid = lax.axis_index("s") * info.num_cores + lax.axis_index("c")
        base = wid * per_w
        pltpu.sync_copy(inv_hbm.at[wid], idx_v)
        bufs = (rows0, rows1)
        loads = [None] * n_it
        scats = [None] * n_it
        loads[0] = pltpu.async_copy(
            x_hbm.at[pl.ds(base, ch)], bufs[0], lsem)
        for i in range(n_it):
            cur = bufs[i % 2]
            loads[i].wait()
            if i + 1 < n_it:
                if i >= 1:
                    scats[i - 1].wait()
                loads[i + 1] = pltpu.async_copy(
                    x_hbm.at[pl.ds(base + (i + 1) * ch, ch)],
                    bufs[(i + 1) % 2], lsem)
            scats[i] = pltpu.async_copy(
                cur, out_hbm.at[idx_v.at[i]], ssem)
        scats[n_it - 1].wait()
        if n_it >= 2:
            scats[n_it - 2].wait()

    return k(x, inv3)


def _row_gather(table, idx):
    R = idx.shape[0]
    D = table.shape[1]
    info = plsc.get_sparse_core_info()
    nw = info.num_cores * info.num_subcores
    per_w = R // nw
    ch = 48 if per_w % 48 == 0 else (32 if per_w % 32 == 0 else per_w)
    n_it = per_w // ch
    mesh = plsc.VectorSubcoreMesh(core_axis_name="c", subcore_axis_name="s")

    @functools.partial(
        pl.kernel,
        mesh=mesh,
        out_type=jax.ShapeDtypeStruct((R, D), table.dtype),
        scratch_types=[
            pltpu.VMEM((per_w,), jnp.int32),
            pltpu.VMEM((ch, D), table.dtype),
            pltpu.VMEM((ch, D), table.dtype),
            pltpu.SemaphoreType.DMA,
            pltpu.SemaphoreType.DMA,
        ],
    )
    def k(table_hbm, idx_hbm, out_hbm, idx_v, rows0, rows1, gsem, ssem):
        wid = lax.axis_index("s") * info.num_cores + lax.axis_index("c")
        base = wid * per_w
        pltpu.sync_copy(idx_hbm.at[pl.ds(base, per_w)], idx_v)
        bufs = (rows0, rows1)
        gathers = [None] * n_it
        stores = [None] * n_it
        gathers[0] = pltpu.async_copy(
            table_hbm.at[idx_v.at[pl.ds(0, ch)]], bufs[0], gsem)
        for i in range(n_it):
            cur = bufs[i % 2]
            gathers[i].wait()
            if i + 1 < n_it:
                if i >= 1:
                    stores[i - 1].wait()
                gathers[i + 1] = pltpu.async_copy(
                    table_hbm.at[idx_v.at[pl.ds((i + 1) * ch, ch)]],
                    bufs[(i + 1) % 2], gsem)
            stores[i] = pltpu.async_copy(
                cur, out_hbm.at[pl.ds(base + i * ch, ch)], ssem)
        stores[n_it - 1].wait()
        if n_it >= 2:
            stores[n_it - 2].wait()

    return k(table, idx)


def _grouped_ffn(xs, W1, b1, W2, b2, block_expert, nb_tot):
    P, D = xs.shape
    F = W1.shape[2]

    def body(be_ref, nbt_ref, xs_ref, w1_ref, b1_ref, w2_ref, b2_ref, o_ref):
        @pl.when(pl.program_id(0) < nbt_ref[0])
        def _():
            h = jnp.dot(xs_ref[...], w1_ref[0],
                        preferred_element_type=jnp.float32) + b1_ref[0, 0]
            h = jax.nn.gelu(h)
            o_ref[...] = jnp.dot(h, w2_ref[0],
                                 preferred_element_type=jnp.float32) + b2_ref[0, 0]

    grid_spec = pltpu.PrefetchScalarGridSpec(
        num_scalar_prefetch=2,
        grid=(P // _BT,),
        in_specs=[
            pl.BlockSpec((_BT, D), lambda i, be, nbt: (i, 0)),
            pl.BlockSpec((1, D, F), lambda i, be, nbt: (be[i], 0, 0)),
            pl.BlockSpec((1, 1, F), lambda i, be, nbt: (be[i], 0, 0)),
            pl.BlockSpec((1, F, D), lambda i, be, nbt: (be[i], 0, 0)),
            pl.BlockSpec((1, 1, D), lambda i, be, nbt: (be[i], 0, 0)),
        ],
        out_specs=pl.BlockSpec((_BT, D), lambda i, be, nbt: (i, 0)),
    )
    En = W1.shape[0]
    return pl.pallas_call(
        body,
        grid_spec=grid_spec,
        out_shape=jax.ShapeDtypeStruct((P, D), jnp.float32),
    )(block_expert, nb_tot, xs, W1, b1.reshape(En, 1, F), W2,
      b2.reshape(En, 1, D))


def kernel(x, Wg, bg, W1, b1, W2, b2):
    T, D = x.shape
    En = Wg.shape[1]
    nbmax = T // _BT + En
    P = nbmax * _BT
    info = plsc.get_sparse_core_info()
    nw = info.num_cores * info.num_subcores
    ch = 32
    e_o, r_o, c_o = _gate_route(x, Wg, bg)
    p_o, block_expert, nb_tot = _pack_route(e_o, r_o, c_o, 128)
    inv = p_o.reshape(T)
    inv3 = p_o.reshape(nw, T // (nw * ch), ch)
    xs = _row_scatter(x, inv3, P)
    ys = _grouped_ffn(xs, W1, b1, W2, b2, block_expert, nb_tot)
    return _row_gather(ys, inv)
